# Initial kernel scaffold; baseline (speedup 1.0000x reference)
import jax
import jax.numpy as jnp
from jax.experimental import pallas as pl


def kernel(x, emb):
    raise NotImplementedError("write your pallas kernel here")


# speedup vs baseline: 1.3064x; 1.3064x over previous
import functools
import math

import jax
import jax.numpy as jnp
from jax import lax
from jax.experimental import pallas as pl
from jax.experimental.pallas import tpu as pltpu
from jax.experimental.pallas import tpu_sc as plsc

D_MODEL = 32
_SCALE = float(math.sqrt(32.0))

_NC = 2
_NS = 16
_NW = _NC * _NS
_LANES = 16


def _build_lookup(n_total, vocab, d, chunk):
    n_per = n_total // _NW
    n_chunks = n_per // chunk
    assert n_per * _NW == n_total and n_chunks * chunk == n_per

    mesh = plsc.VectorSubcoreMesh(core_axis_name="c", subcore_axis_name="s")

    @functools.partial(
        pl.kernel,
        out_type=jax.ShapeDtypeStruct((n_total, d), jnp.float32),
        mesh=mesh,
        compiler_params=pltpu.CompilerParams(use_tc_tiling_on_sc=False),
        scratch_types=[
            pltpu.VMEM((n_per,), jnp.int32),
            pltpu.VMEM((chunk, d), jnp.float32),
            pltpu.SemaphoreType.DMA,
        ],
    )
    def lookup(idx_hbm, table_hbm, out_hbm, idx_v, rows_v, sem):
        wid = lax.axis_index("s") * _NC + lax.axis_index("c")
        base = wid * n_per
        pltpu.sync_copy(idx_hbm.at[pl.ds(base, n_per)], idx_v)

        def scale_row(i, _):
            for j in range(d // _LANES):
                sl = pl.ds(j * _LANES, _LANES)
                rows_v[i, sl] = rows_v[i, sl] * _SCALE
            return 0

        for g in range(n_chunks):
            off = g * chunk
            pltpu.async_copy(
                table_hbm.at[idx_v.at[pl.ds(off, chunk)]], rows_v, sem
            ).wait()
            lax.fori_loop(0, chunk, scale_row, 0)
            pltpu.sync_copy(rows_v, out_hbm.at[pl.ds(base + off, chunk)])

    return lookup


def kernel(x, emb):
    b, l = x.shape
    vocab, d = emb.shape
    xf = x.reshape(-1).astype(jnp.int32)
    lookup = _build_lookup(b * l, vocab, d, chunk=1024)
    out = lookup(xf, emb)
    return out.reshape(b, l, d)


# speedup vs baseline: 1.4765x; 1.1302x over previous
import functools
import math

import jax
import jax.numpy as jnp
from jax import lax
from jax.experimental import pallas as pl
from jax.experimental.pallas import tpu as pltpu
from jax.experimental.pallas import tpu_sc as plsc

D_MODEL = 32
_SCALE = float(math.sqrt(32.0))

_NC = 2
_NS = 16
_NW = _NC * _NS
_LANES = 16


def _build_lookup(n_total, vocab, d, chunk):
    n_per = n_total // _NW
    n_chunks = n_per // chunk
    assert n_per * _NW == n_total and n_chunks * chunk == n_per

    mesh = plsc.VectorSubcoreMesh(core_axis_name="c", subcore_axis_name="s")

    unroll = 4

    @functools.partial(
        pl.kernel,
        out_type=jax.ShapeDtypeStruct((n_total, d), jnp.float32),
        mesh=mesh,
        compiler_params=pltpu.CompilerParams(use_tc_tiling_on_sc=False),
        scratch_types=[
            pltpu.VMEM((n_per,), jnp.int32),
            pltpu.VMEM((chunk, d), jnp.float32),
            pltpu.VMEM((chunk, d), jnp.float32),
            pltpu.SemaphoreType.DMA,
            pltpu.SemaphoreType.DMA,
            pltpu.SemaphoreType.DMA,
            pltpu.SemaphoreType.DMA,
        ],
    )
    def lookup(idx_hbm, table_hbm, out_hbm, idx_v, rows0, rows1, g0, g1, o0, o1):
        wid = lax.axis_index("s") * _NC + lax.axis_index("c")
        base = wid * n_per
        pltpu.sync_copy(idx_hbm.at[pl.ds(base, n_per)], idx_v)

        bufs = (rows0, rows1)
        gsems = (g0, g1)
        osems = (o0, o1)

        def start_gather(g):
            b = g % 2
            return pltpu.async_copy(
                table_hbm.at[idx_v.at[pl.ds(g * chunk, chunk)]], bufs[b], gsems[b]
            )

        def scale_rows(buf):
            def body(i, _):
                for u in range(unroll):
                    for j in range(d // _LANES):
                        sl = pl.ds(j * _LANES, _LANES)
                        buf[i * unroll + u, sl] = buf[i * unroll + u, sl] * _SCALE
                return 0

            lax.fori_loop(0, chunk // unroll, body, 0)

        gcp = start_gather(0)
        prev_out = None
        for g in range(n_chunks):
            b = g % 2
            gcp.wait()
            if g + 1 < n_chunks:
                if prev_out is not None:
                    prev_out.wait()
                gcp = start_gather(g + 1)
            scale_rows(bufs[b])
            ocp = pltpu.async_copy(
                bufs[b], out_hbm.at[pl.ds(base + g * chunk, chunk)], osems[b]
            )
            if prev_out is not None and g + 1 >= n_chunks:
                prev_out.wait()
            prev_out = ocp
        prev_out.wait()

    return lookup


def kernel(x, emb):
    b, l = x.shape
    vocab, d = emb.shape
    xf = x.reshape(-1).astype(jnp.int32)
    lookup = _build_lookup(b * l, vocab, d, chunk=1024)
    out = lookup(xf, emb)
    return out.reshape(b, l, d)
